# Initial kernel scaffold; baseline (speedup 1.0000x reference)
import jax
import jax.numpy as jnp
from jax.experimental import pallas as pl


def kernel(x, edge_index, W1, b1, Wg, att_src, att_dst, bg, W3, b3, Wm1, bm1, gamma, beta, Wm2, bm2, Wu, bu, type_unc):
    raise NotImplementedError("write your pallas kernel here")


# speedup vs baseline: 26.8918x; 26.8918x over previous
import functools

import jax
import jax.numpy as jnp
from jax import lax
from jax.experimental import pallas as pl
from jax.experimental.pallas import tpu as pltpu
from jax.experimental.pallas import tpu_sc as plsc

N = 10000
NP = 10240
E = 320000
D = 128
H = 4
C = 32
T = 10
TEMP = 0.7

NC = 2
NS = 16
NW = NC * NS
EPW = E // NW
RPT = NP // NS

KD = 80
NITD = EPW // KD
EPWP = 10240
KE = 128
NITE = EPWP // KE
SEGE = 8
KB = 64
NITB = EPWP // KB
SEGB = 8

_f32 = jnp.float32
_i32 = jnp.int32


def _wid():
    cid = lax.axis_index("c")
    sid = lax.axis_index("s")
    return cid, sid, cid * NS + sid


_SC_MESH = dict(
    mesh=plsc.VectorSubcoreMesh(core_axis_name="c", subcore_axis_name="s",
                                num_cores=NC, num_subcores=NS),
    compiler_params=pltpu.CompilerParams(needs_layout_passes=False),
)


@functools.partial(
    pl.kernel,
    out_type=jax.ShapeDtypeStruct((NC, NP, 8), _f32),
    scratch_types=[
        pltpu.VMEM((NITD, KD), _i32),
        pltpu.VMEM((KD, 8), _f32),
        pltpu.VMEM_SHARED((NP, 8), _f32),
        pltpu.SemaphoreType.DMA,
    ],
    **_SC_MESH,
)
def _deg_kernel(dsts_hbm, ones_hbm, zeros_hbm, deg_hbm,
                didx_v, ones_v, acc_sh, sem):
    cid, sid, wid = _wid()
    pltpu.sync_copy(dsts_hbm.at[wid], didx_v)
    pltpu.sync_copy(ones_hbm, ones_v)
    rows = pl.ds(sid * RPT, RPT)
    pltpu.sync_copy(zeros_hbm.at[rows], acc_sh.at[rows])
    plsc.subcore_barrier()

    pltpu.async_copy(ones_v, acc_sh.at[didx_v.at[0]], sem, add=True)

    def body(t, carry):
        @pl.when(t + 1 < NITD)
        def _():
            pltpu.async_copy(ones_v, acc_sh.at[didx_v.at[t + 1]], sem,
                             add=True)
        pltpu.make_async_copy(ones_v, acc_sh.at[didx_v.at[0]], sem).wait()
        return carry

    lax.fori_loop(0, NITD, body, 0)
    plsc.subcore_barrier()
    pltpu.sync_copy(acc_sh.at[rows], deg_hbm.at[cid, rows])


@functools.partial(
    pl.kernel,
    out_type=jax.ShapeDtypeStruct((NC, NP, D), _f32),
    scratch_types=[
        pltpu.VMEM((SEGE, KE), _i32),
        pltpu.VMEM((SEGE, KE), _i32),
        pltpu.VMEM((KE, D), _f32),
        pltpu.VMEM((KE, D), _f32),
        pltpu.VMEM_SHARED((NP, D), _f32),
        pltpu.SemaphoreType.DMA,
        pltpu.SemaphoreType.DMA,
    ],
    **_SC_MESH,
)
def _gcn_kernel(xw_hbm, srcs_hbm, dsts_hbm, zeros_hbm, out_hbm,
                sidx_v, didx_v, rows0, rows1, acc_sh, gsem0, gsem1):
    cid, sid, wid = _wid()
    rows = pl.ds(sid * RPT, RPT)
    pltpu.sync_copy(zeros_hbm.at[rows], acc_sh.at[rows])
    plsc.subcore_barrier()

    rowsb = (rows0, rows1)
    gsem = (gsem0, gsem1)

    def cpseg(s):
        s = pl.multiple_of(s, SEGE)
        pltpu.sync_copy(srcs_hbm.at[wid, pl.ds(s, SEGE)], sidx_v)
        pltpu.sync_copy(dsts_hbm.at[wid, pl.ds(s, SEGE)], didx_v)

    def issue(jl, b):
        pltpu.async_copy(xw_hbm.at[sidx_v.at[jl]], rowsb[b], gsem[b])

    def drain(b):
        pltpu.make_async_copy(xw_hbm.at[sidx_v.at[0]], rowsb[b],
                              gsem[b]).wait()

    cpseg(0)
    issue(0, 0)
    issue(1, 1)

    def body(t, carry):
        j = 2 * t
        jl = lax.rem(j, SEGE)
        drain(0)
        pltpu.sync_copy(rows0, acc_sh.at[didx_v.at[jl]], add=True)
        drain(1)
        pltpu.sync_copy(rows1, acc_sh.at[didx_v.at[jl + 1]], add=True)

        @pl.when(j + 2 < NITE)
        def _():
            @pl.when(lax.rem(j + 2, SEGE) == 0)
            def _():
                cpseg(j + 2)
            issue(lax.rem(j + 2, SEGE), 0)

        @pl.when(j + 3 < NITE)
        def _():
            issue(lax.rem(j + 3, SEGE), 1)
        return carry

    lax.fori_loop(0, NITE // 2, body, 0)
    plsc.subcore_barrier()
    pltpu.sync_copy(acc_sh.at[rows], out_hbm.at[cid, rows])


@functools.partial(
    pl.kernel,
    out_type=jax.ShapeDtypeStruct((NW, NITB, KB, 16), _f32),
    scratch_types=[
        pltpu.VMEM((SEGB, KB), _i32),
        pltpu.VMEM((SEGB, KB), _i32),
        pltpu.VMEM((NP * 8,), _f32),
        pltpu.VMEM((KB, 16), _f32),
        pltpu.VMEM((KB, 16), _f32),
        pltpu.SemaphoreType.DMA,
        pltpu.SemaphoreType.DMA,
    ],
    **_SC_MESH,
)
def _gat_att_kernel(alsd_hbm, srcs_hbm, dsts_hbm, zk_hbm, ee_hbm,
                    sidx_v, didx_v, tab_v, ee0, ee1, esem0, esem1):
    cid, sid, wid = _wid()
    pltpu.sync_copy(alsd_hbm, tab_v)
    pltpu.sync_copy(zk_hbm, ee0)
    pltpu.sync_copy(zk_hbm, ee1)

    eeb = (ee0, ee1)
    esem = (esem0, esem1)
    iota16 = lax.broadcasted_iota(_i32, (16,), 0)

    def cpseg(s):
        s = pl.multiple_of(s, SEGB)
        pltpu.sync_copy(srcs_hbm.at[wid, pl.ds(s, SEGB)], sidx_v)
        pltpu.sync_copy(dsts_hbm.at[wid, pl.ds(s, SEGB)], didx_v)

    def process(j, jl, b):
        @pl.when(j >= 2)
        def _():
            pltpu.make_async_copy(eeb[b], ee_hbm.at[0, 0], esem[b]).wait()
        for g in range(KB // 16):
            sv = sidx_v[jl, pl.ds(g * 16, 16)] * 8
            dv = didx_v[jl, pl.ds(g * 16, 16)] * 8
            rv = iota16 + (g * 16)
            for h in range(H):
                hs = jnp.full((16,), h, _i32)
                va = (plsc.load_gather(tab_v, [sv + h])
                      + plsc.load_gather(tab_v, [dv + (h + 4)]))
                ee = jnp.exp(jnp.where(va >= 0.0, va, 0.2 * va))
                plsc.store_scatter(eeb[b], [rv, hs], ee)
        pltpu.async_copy(eeb[b], ee_hbm.at[wid, j], esem[b])

    cpseg(0)

    def body(t, carry):
        j = 2 * t
        jl = lax.rem(j, SEGB)
        process(j, jl, 0)
        process(j + 1, jl + 1, 1)

        @pl.when(j + 2 < NITB)
        def _():
            @pl.when(lax.rem(j + 2, SEGB) == 0)
            def _():
                cpseg(j + 2)
        return carry

    lax.fori_loop(0, NITB // 2, body, 0)
    pltpu.make_async_copy(ee0, ee_hbm.at[0, 0], esem0).wait()
    pltpu.make_async_copy(ee1, ee_hbm.at[0, 0], esem1).wait()


@functools.partial(
    pl.kernel,
    out_type=jax.ShapeDtypeStruct((NC, NP, 16), _f32),
    scratch_types=[
        pltpu.VMEM((SEGB, KB), _i32),
        pltpu.VMEM((KB, 16), _f32),
        pltpu.VMEM((KB, 16), _f32),
        pltpu.VMEM_SHARED((NP, 16), _f32),
        pltpu.SemaphoreType.DMA,
        pltpu.SemaphoreType.DMA,
    ],
    **_SC_MESH,
)
def _den_kernel(ee_hbm, dsts_hbm, zeros16_hbm, den_hbm,
                didx_v, ee0, ee1, dacc_sh, gsem0, gsem1):
    cid, sid, wid = _wid()
    rows = pl.ds(sid * RPT, RPT)
    pltpu.sync_copy(zeros16_hbm.at[rows], dacc_sh.at[rows])
    plsc.subcore_barrier()

    eeb = (ee0, ee1)
    gsem = (gsem0, gsem1)

    def cpseg(s):
        s = pl.multiple_of(s, SEGB)
        pltpu.sync_copy(dsts_hbm.at[wid, pl.ds(s, SEGB)], didx_v)

    def issue(j, b):
        pltpu.async_copy(ee_hbm.at[wid, j], eeb[b], gsem[b])

    def drain(b):
        pltpu.make_async_copy(ee_hbm.at[0, 0], eeb[b], gsem[b]).wait()

    cpseg(0)
    issue(0, 0)
    issue(1, 1)

    def body(t, carry):
        j = 2 * t
        jl = lax.rem(j, SEGB)
        drain(0)
        pltpu.sync_copy(ee0, dacc_sh.at[didx_v.at[jl]], add=True)
        drain(1)
        pltpu.sync_copy(ee1, dacc_sh.at[didx_v.at[jl + 1]], add=True)

        @pl.when(j + 2 < NITB)
        def _():
            @pl.when(lax.rem(j + 2, SEGB) == 0)
            def _():
                cpseg(j + 2)
            issue(j + 2, 0)

        @pl.when(j + 3 < NITB)
        def _():
            issue(j + 3, 1)
        return carry

    lax.fori_loop(0, NITB // 2, body, 0)
    plsc.subcore_barrier()
    pltpu.sync_copy(dacc_sh.at[rows], den_hbm.at[cid, rows])


@functools.partial(
    pl.kernel,
    out_type=jax.ShapeDtypeStruct((NC, NP, D), _f32),
    scratch_types=[
        pltpu.VMEM((SEGB, KB), _i32),
        pltpu.VMEM((SEGB, KB), _i32),
        pltpu.VMEM((KB, D), _f32),
        pltpu.VMEM((KB, D), _f32),
        pltpu.VMEM((KB, 16), _f32),
        pltpu.VMEM((KB, 16), _f32),
        pltpu.VMEM_SHARED((NP, D), _f32),
        pltpu.SemaphoreType.DMA,
        pltpu.SemaphoreType.DMA,
    ],
    **_SC_MESH,
)
def _gat_agg_kernel(xw2_hbm, ee_hbm, srcs_hbm, dsts_hbm, zeros_hbm,
                    num_hbm,
                    sidx_v, didx_v, rows0, rows1, ee0, ee1,
                    nacc_sh, gsem0, gsem1):
    cid, sid, wid = _wid()
    rows = pl.ds(sid * RPT, RPT)
    pltpu.sync_copy(zeros_hbm.at[rows], nacc_sh.at[rows])
    plsc.subcore_barrier()

    rowsb = (rows0, rows1)
    eeb = (ee0, ee1)
    gsem = (gsem0, gsem1)

    def cpseg(s):
        s = pl.multiple_of(s, SEGB)
        pltpu.sync_copy(srcs_hbm.at[wid, pl.ds(s, SEGB)], sidx_v)
        pltpu.sync_copy(dsts_hbm.at[wid, pl.ds(s, SEGB)], didx_v)

    def issue(jl, j, b):
        pltpu.async_copy(xw2_hbm.at[sidx_v.at[jl]], rowsb[b], gsem[b])
        pltpu.async_copy(ee_hbm.at[wid, j], eeb[b], gsem[b])

    def drain(b):
        pltpu.make_async_copy(xw2_hbm.at[sidx_v.at[0]], rowsb[b],
                              gsem[b]).wait()
        pltpu.make_async_copy(ee_hbm.at[0, 0], eeb[b], gsem[b]).wait()

    def process(jl, b):
        rb = rowsb[b]

        def scale(i, carry):
            eer = eeb[b][i, pl.ds(0, 16)]
            for h in range(H):
                s = eer[h]
                for q in range(2):
                    sl = pl.ds(h * 32 + q * 16, 16)
                    rb[i, sl] = rb[i, sl] * s
            return carry

        lax.fori_loop(0, KB, scale, 0)
        pltpu.sync_copy(rb, nacc_sh.at[didx_v.at[jl]], add=True)

    cpseg(0)
    issue(0, 0, 0)
    issue(1, 1, 1)

    def body(t, carry):
        j = 2 * t
        jl = lax.rem(j, SEGB)
        drain(0)
        process(jl, 0)
        drain(1)
        process(jl + 1, 1)

        @pl.when(j + 2 < NITB)
        def _():
            @pl.when(lax.rem(j + 2, SEGB) == 0)
            def _():
                cpseg(j + 2)
            issue(lax.rem(j + 2, SEGB), j + 2, 0)

        @pl.when(j + 3 < NITB)
        def _():
            issue(lax.rem(j + 3, SEGB), j + 3, 1)
        return carry

    lax.fori_loop(0, NITB // 2, body, 0)
    plsc.subcore_barrier()
    pltpu.sync_copy(nacc_sh.at[rows], num_hbm.at[cid, rows])


_B = 512
_GRID = NP // _B


def _full_spec(shape):
    return pl.BlockSpec(shape, lambda i: (0,) * len(shape))


def _row_spec(cols):
    return pl.BlockSpec((_B, cols), lambda i: (i, 0))


def _pair_spec(cols):
    return pl.BlockSpec((NC, _B, cols), lambda i: (0, i, 0))


def _tc_call(body, in_specs, out_specs, out_shapes):
    return pl.pallas_call(
        body,
        grid=(_GRID,),
        in_specs=in_specs,
        out_specs=out_specs,
        out_shape=out_shapes,
    )


def _k2_body(x_ref, w_ref, dg_ref, xw_ref, dv_ref):
    dv8 = lax.rsqrt(1.0 + dg_ref[0] + dg_ref[1])
    xw = jnp.dot(x_ref[...], w_ref[...], preferred_element_type=_f32)
    xw_ref[...] = xw * dv8[:, 0:1]
    dv_ref[...] = dv8


def _k4_body(a_ref, xwp_ref, dv_ref, b1_ref, wg_ref, asd_ref,
             xw2_ref, alsd_ref):
    dv = dv_ref[...][:, 0:1]
    h1 = jnp.maximum(
        dv * (a_ref[0] + a_ref[1] + xwp_ref[...]) + b1_ref[...], 0.0)
    xw2 = jnp.dot(h1, wg_ref[...], preferred_element_type=_f32)
    xw2_ref[...] = xw2
    alsd_ref[...] = jnp.dot(xw2, asd_ref[...], preferred_element_type=_f32)


def _k6_body(n_ref, q_ref, alsd_ref, xw2_ref,
             bg_ref, w3_ref, dv_ref, out_ref):
    al = alsd_ref[...]
    a = al[:, 0:4] + al[:, 4:8]
    ees = jnp.exp(jnp.where(a >= 0.0, a, 0.2 * a))
    den4 = q_ref[0][:, 0:4] + q_ref[1][:, 0:4] + ees + 1e-16
    hh = lax.broadcasted_iota(_i32, (4, D), 0)
    cc = lax.broadcasted_iota(_i32, (4, D), 1)
    sel = (cc // C == hh).astype(_f32)
    eef = jnp.dot(ees, sel, preferred_element_type=_f32)
    denf = jnp.dot(den4, sel, preferred_element_type=_f32)
    num = n_ref[0] + n_ref[1] + eef * xw2_ref[...]
    h2 = jnp.maximum(num / denf + bg_ref[...], 0.0)
    out_ref[...] = (jnp.dot(h2, w3_ref[...], preferred_element_type=_f32)
                    * dv_ref[...][:, 0:1])


_RS = float((1.0 + 1e-5) ** -0.5)


def _k8_body(r_ref, xwp_ref, dv_ref, b3_ref, wm1_ref, bm1_ref,
             gam_ref, bet_ref, wm2_ref, bm2_ref, wu_ref, bu_ref, tv_ref,
             p_ref, tu_ref):
    dv = dv_ref[...][:, 0:1]
    h3 = jnp.maximum(
        dv * (r_ref[0] + r_ref[1] + xwp_ref[...]) + b3_ref[...], 0.0)

    def mlp(h):
        z = jnp.dot(h, wm1_ref[...], preferred_element_type=_f32) + bm1_ref[...]
        z = z * _RS * gam_ref[...] + bet_ref[...]
        z = jnp.maximum(z, 0.0)
        return jnp.dot(z, wm2_ref[...], preferred_element_type=_f32) + bm2_ref[...]

    def softmax(logits):
        l = logits / TEMP
        m = jnp.max(l, axis=1, keepdims=True)
        ex = jnp.exp(l - m)
        return ex / jnp.sum(ex, axis=1, keepdims=True)

    tv = tv_ref[...]
    p1 = softmax(mlp(h3))
    us = jnp.dot(p1 * tv, wu_ref[...], preferred_element_type=_f32) + bu_ref[...]
    p2 = softmax(mlp(h3 + us))
    p_ref[...] = p2[:, 0:T]
    tu_ref[...] = jnp.sum(p2 * tv, axis=1, keepdims=True)


def kernel(x, edge_index, W1, b1, Wg, att_src, att_dst, bg, W3, b3,
           Wm1, bm1, gamma, beta, Wm2, bm2, Wu, bu, type_unc):
    pad = jnp.full((NW, EPWP - EPW), N, _i32)
    srcp = jnp.concatenate([edge_index[0].reshape(NW, EPW), pad], axis=1)
    dstp = jnp.concatenate([edge_index[1].reshape(NW, EPW), pad], axis=1)
    srcsE = srcp.reshape(NW, NITE, KE)
    dstsE = dstp.reshape(NW, NITE, KE)
    srcsB = srcp.reshape(NW, NITB, KB)
    dstsB = dstp.reshape(NW, NITB, KB)
    dsts80 = edge_index[1].reshape(NW, NITD, KD)
    xp = jnp.zeros((NP, D), _f32).at[:N].set(x)

    zerosND = jnp.zeros((NP, D), _f32)
    zerosN8 = jnp.zeros((NP, 8), _f32)
    zerosN16 = jnp.zeros((NP, 16), _f32)
    zerosKB16 = jnp.zeros((KB, 16), _f32)
    onesK8 = jnp.ones((KD, 8), _f32)

    eye = jnp.eye(H, dtype=_f32)
    Asd = jnp.concatenate(
        [(att_src[:, :, None] * eye[:, None, :]).reshape(D, H),
         (att_dst[:, :, None] * eye[:, None, :]).reshape(D, H)], axis=1)

    Wm2p = jnp.zeros((D, D), _f32).at[:, :T].set(Wm2)
    bm2p = jnp.full((1, D), -1e30, _f32).at[0, :T].set(bm2)
    Wup = jnp.zeros((D, D), _f32).at[:T, :].set(Wu)
    tvp = jnp.zeros((1, D), _f32).at[0, :T].set(type_unc)

    b1r = b1.reshape(1, D)
    bgr = bg.reshape(1, D)
    b3r = b3.reshape(1, D)
    bm1r = bm1.reshape(1, D)
    gammar = gamma.reshape(1, D)
    betar = beta.reshape(1, D)
    bur = bu.reshape(1, D)

    deg = _deg_kernel(dsts80, onesK8, zerosN8)

    xw1p, dinv8 = _tc_call(
        _k2_body,
        [_row_spec(D), _full_spec((D, D)), _pair_spec(8)],
        [_row_spec(D), _row_spec(8)],
        [jax.ShapeDtypeStruct((NP, D), _f32), jax.ShapeDtypeStruct((NP, 8), _f32)],
    )(xp, W1, deg)

    agg1 = _gcn_kernel(xw1p, srcsE, dstsE, zerosND)

    xw2, alsd = _tc_call(
        _k4_body,
        [_pair_spec(D), _row_spec(D), _row_spec(8),
         _full_spec((1, D)), _full_spec((D, D)), _full_spec((D, 2 * H))],
        [_row_spec(D), _row_spec(2 * H)],
        [jax.ShapeDtypeStruct((NP, D), _f32),
         jax.ShapeDtypeStruct((NP, 2 * H), _f32)],
    )(agg1, xw1p, dinv8, b1r, Wg, Asd)

    ee = _gat_att_kernel(alsd.reshape(NP * 8), srcsB, dstsB, zerosKB16)
    q = _den_kernel(ee, dstsB, zerosN16)
    num = _gat_agg_kernel(xw2, ee, srcsB, dstsB, zerosND)

    xw3p = _tc_call(
        _k6_body,
        [_pair_spec(D), _pair_spec(16),
         _row_spec(2 * H), _row_spec(D),
         _full_spec((1, D)), _full_spec((D, D)), _row_spec(8)],
        _row_spec(D),
        jax.ShapeDtypeStruct((NP, D), _f32),
    )(num, q, alsd, xw2, bgr, W3, dinv8)

    agg3 = _gcn_kernel(xw3p, srcsE, dstsE, zerosND)

    p, tu = _tc_call(
        _k8_body,
        [_pair_spec(D), _row_spec(D), _row_spec(8),
         _full_spec((1, D)), _full_spec((D, D)), _full_spec((1, D)),
         _full_spec((1, D)), _full_spec((1, D)), _full_spec((D, D)),
         _full_spec((1, D)), _full_spec((D, D)), _full_spec((1, D)),
         _full_spec((1, D))],
        [_row_spec(T), _row_spec(1)],
        [jax.ShapeDtypeStruct((NP, T), _f32), jax.ShapeDtypeStruct((NP, 1), _f32)],
    )(agg3, xw3p, dinv8, b3r, Wm1, bm1r, gammar, betar, Wm2p, bm2p,
      Wup, bur, tvp)

    return (p[:N], tu[:N])


# speedup vs baseline: 27.3056x; 1.0154x over previous
import functools

import jax
import jax.numpy as jnp
from jax import lax
from jax.experimental import pallas as pl
from jax.experimental.pallas import tpu as pltpu
from jax.experimental.pallas import tpu_sc as plsc

N = 10000
NP = 10240
E = 320000
D = 128
H = 4
C = 32
T = 10
TEMP = 0.7

NC = 2
NS = 16
NW = NC * NS
EPW = E // NW
RPT = NP // NS

KD = 80
NITD = EPW // KD
EPWP = 10240
KE = 128
NITE = EPWP // KE
SEGE = 8
KB = 64
NITB = EPWP // KB
SEGB = 8

_f32 = jnp.float32
_i32 = jnp.int32


def _wid():
    cid = lax.axis_index("c")
    sid = lax.axis_index("s")
    return cid, sid, cid * NS + sid


_SC_MESH = dict(
    mesh=plsc.VectorSubcoreMesh(core_axis_name="c", subcore_axis_name="s",
                                num_cores=NC, num_subcores=NS),
    compiler_params=pltpu.CompilerParams(needs_layout_passes=False),
)


@functools.partial(
    pl.kernel,
    out_type=jax.ShapeDtypeStruct((NC, NP, 8), _f32),
    scratch_types=[
        pltpu.VMEM((NITD, KD), _i32),
        pltpu.VMEM((KD, 8), _f32),
        pltpu.VMEM_SHARED((NP, 8), _f32),
        pltpu.SemaphoreType.DMA,
    ],
    **_SC_MESH,
)
def _deg_kernel(dsts_hbm, ones_hbm, zeros_hbm, deg_hbm,
                didx_v, ones_v, acc_sh, sem):
    cid, sid, wid = _wid()
    pltpu.sync_copy(dsts_hbm.at[wid], didx_v)
    pltpu.sync_copy(ones_hbm, ones_v)
    rows = pl.ds(sid * RPT, RPT)
    pltpu.sync_copy(zeros_hbm.at[rows], acc_sh.at[rows])
    plsc.subcore_barrier()

    DEPTH = 2
    for t0 in range(DEPTH):
        pltpu.async_copy(ones_v, acc_sh.at[didx_v.at[t0]], sem, add=True)

    def body(t, carry):
        @pl.when(t + DEPTH < NITD)
        def _():
            pltpu.async_copy(ones_v, acc_sh.at[didx_v.at[t + DEPTH]], sem,
                             add=True)
        pltpu.make_async_copy(ones_v, acc_sh.at[didx_v.at[0]], sem).wait()
        return carry

    lax.fori_loop(0, NITD, body, 0)
    plsc.subcore_barrier()
    pltpu.sync_copy(acc_sh.at[rows], deg_hbm.at[cid, rows])


@functools.partial(
    pl.kernel,
    out_type=jax.ShapeDtypeStruct((NC, NP, D), _f32),
    scratch_types=[
        pltpu.VMEM((SEGE, KE), _i32),
        pltpu.VMEM((SEGE, KE), _i32),
        pltpu.VMEM((KE, D), _f32),
        pltpu.VMEM((KE, D), _f32),
        pltpu.VMEM_SHARED((NP, D), _f32),
        pltpu.SemaphoreType.DMA,
        pltpu.SemaphoreType.DMA,
    ],
    **_SC_MESH,
)
def _gcn_kernel(xw_hbm, srcs_hbm, dsts_hbm, zeros_hbm, out_hbm,
                sidx_v, didx_v, rows0, rows1, acc_sh, gsem0, gsem1):
    cid, sid, wid = _wid()
    rows = pl.ds(sid * RPT, RPT)
    pltpu.sync_copy(zeros_hbm.at[rows], acc_sh.at[rows])
    plsc.subcore_barrier()

    rowsb = (rows0, rows1)
    gsem = (gsem0, gsem1)

    def cpseg(s):
        s = pl.multiple_of(s, SEGE)
        pltpu.sync_copy(srcs_hbm.at[wid, pl.ds(s, SEGE)], sidx_v)
        pltpu.sync_copy(dsts_hbm.at[wid, pl.ds(s, SEGE)], didx_v)

    def issue(jl, b):
        pltpu.async_copy(xw_hbm.at[sidx_v.at[jl]], rowsb[b], gsem[b])

    def drain(b):
        pltpu.make_async_copy(xw_hbm.at[sidx_v.at[0]], rowsb[b],
                              gsem[b]).wait()

    cpseg(0)
    issue(0, 0)
    issue(1, 1)

    def body(t, carry):
        j = 2 * t
        jl = lax.rem(j, SEGE)
        drain(0)
        pltpu.sync_copy(rows0, acc_sh.at[didx_v.at[jl]], add=True)
        drain(1)
        pltpu.sync_copy(rows1, acc_sh.at[didx_v.at[jl + 1]], add=True)

        @pl.when(j + 2 < NITE)
        def _():
            @pl.when(lax.rem(j + 2, SEGE) == 0)
            def _():
                cpseg(j + 2)
            issue(lax.rem(j + 2, SEGE), 0)

        @pl.when(j + 3 < NITE)
        def _():
            issue(lax.rem(j + 3, SEGE), 1)
        return carry

    lax.fori_loop(0, NITE // 2, body, 0)
    plsc.subcore_barrier()
    pltpu.sync_copy(acc_sh.at[rows], out_hbm.at[cid, rows])


@functools.partial(
    pl.kernel,
    out_type=jax.ShapeDtypeStruct((NW, NITB, KB, 16), _f32),
    scratch_types=[
        pltpu.VMEM((SEGB, KB), _i32),
        pltpu.VMEM((SEGB, KB), _i32),
        pltpu.VMEM((NP * 8,), _f32),
        pltpu.VMEM((KB, 16), _f32),
        pltpu.VMEM((KB, 16), _f32),
        pltpu.SemaphoreType.DMA,
        pltpu.SemaphoreType.DMA,
    ],
    **_SC_MESH,
)
def _gat_att_kernel(alsd_hbm, srcs_hbm, dsts_hbm, zk_hbm, ee_hbm,
                    sidx_v, didx_v, tab_v, ee0, ee1, esem0, esem1):
    cid, sid, wid = _wid()
    pltpu.sync_copy(alsd_hbm, tab_v)
    pltpu.sync_copy(zk_hbm, ee0)
    pltpu.sync_copy(zk_hbm, ee1)

    eeb = (ee0, ee1)
    esem = (esem0, esem1)
    iota16 = lax.broadcasted_iota(_i32, (16,), 0)

    def cpseg(s):
        s = pl.multiple_of(s, SEGB)
        pltpu.sync_copy(srcs_hbm.at[wid, pl.ds(s, SEGB)], sidx_v)
        pltpu.sync_copy(dsts_hbm.at[wid, pl.ds(s, SEGB)], didx_v)

    def process(j, jl, b):
        @pl.when(j >= 2)
        def _():
            pltpu.make_async_copy(eeb[b], ee_hbm.at[0, 0], esem[b]).wait()
        for g in range(KB // 16):
            sv = sidx_v[jl, pl.ds(g * 16, 16)] * 8
            dv = didx_v[jl, pl.ds(g * 16, 16)] * 8
            rv = iota16 + (g * 16)
            for h in range(H):
                hs = jnp.full((16,), h, _i32)
                va = (plsc.load_gather(tab_v, [sv + h])
                      + plsc.load_gather(tab_v, [dv + (h + 4)]))
                ee = jnp.exp(jnp.where(va >= 0.0, va, 0.2 * va))
                plsc.store_scatter(eeb[b], [rv, hs], ee)
        pltpu.async_copy(eeb[b], ee_hbm.at[wid, j], esem[b])

    cpseg(0)

    def body(t, carry):
        j = 2 * t
        jl = lax.rem(j, SEGB)
        process(j, jl, 0)
        process(j + 1, jl + 1, 1)

        @pl.when(j + 2 < NITB)
        def _():
            @pl.when(lax.rem(j + 2, SEGB) == 0)
            def _():
                cpseg(j + 2)
        return carry

    lax.fori_loop(0, NITB // 2, body, 0)
    pltpu.make_async_copy(ee0, ee_hbm.at[0, 0], esem0).wait()
    pltpu.make_async_copy(ee1, ee_hbm.at[0, 0], esem1).wait()


@functools.partial(
    pl.kernel,
    out_type=jax.ShapeDtypeStruct((NC, NP, 16), _f32),
    scratch_types=[
        pltpu.VMEM((SEGE, KE), _i32),
        pltpu.VMEM((KE, 16), _f32),
        pltpu.VMEM((KE, 16), _f32),
        pltpu.VMEM_SHARED((NP, 16), _f32),
        pltpu.SemaphoreType.DMA,
        pltpu.SemaphoreType.DMA,
    ],
    **_SC_MESH,
)
def _den_kernel(ee_hbm, dsts_hbm, zeros16_hbm, den_hbm,
                didx_v, ee0, ee1, dacc_sh, gsem0, gsem1):
    cid, sid, wid = _wid()
    rows = pl.ds(sid * RPT, RPT)
    pltpu.sync_copy(zeros16_hbm.at[rows], dacc_sh.at[rows])
    plsc.subcore_barrier()

    eeb = (ee0, ee1)
    gsem = (gsem0, gsem1)

    def cpseg(s):
        s = pl.multiple_of(s, SEGE)
        pltpu.sync_copy(dsts_hbm.at[wid, pl.ds(s, SEGE)], didx_v)

    def issue(j, b):
        pltpu.async_copy(ee_hbm.at[wid, j], eeb[b], gsem[b])

    def drain(b):
        pltpu.make_async_copy(ee_hbm.at[0, 0], eeb[b], gsem[b]).wait()

    cpseg(0)
    issue(0, 0)
    issue(1, 1)

    def body(t, carry):
        j = 2 * t
        jl = lax.rem(j, SEGE)
        drain(0)
        pltpu.sync_copy(ee0, dacc_sh.at[didx_v.at[jl]], add=True)
        drain(1)
        pltpu.sync_copy(ee1, dacc_sh.at[didx_v.at[jl + 1]], add=True)

        @pl.when(j + 2 < NITE)
        def _():
            @pl.when(lax.rem(j + 2, SEGE) == 0)
            def _():
                cpseg(j + 2)
            issue(j + 2, 0)

        @pl.when(j + 3 < NITE)
        def _():
            issue(j + 3, 1)
        return carry

    lax.fori_loop(0, NITE // 2, body, 0)
    plsc.subcore_barrier()
    pltpu.sync_copy(dacc_sh.at[rows], den_hbm.at[cid, rows])


@functools.partial(
    pl.kernel,
    out_type=jax.ShapeDtypeStruct((NC, NP, D), _f32),
    scratch_types=[
        pltpu.VMEM((SEGB, KB), _i32),
        pltpu.VMEM((SEGB, KB), _i32),
        pltpu.VMEM((KB, D), _f32),
        pltpu.VMEM((KB, D), _f32),
        pltpu.VMEM((KB, 16), _f32),
        pltpu.VMEM((KB, 16), _f32),
        pltpu.VMEM_SHARED((NP, D), _f32),
        pltpu.SemaphoreType.DMA,
        pltpu.SemaphoreType.DMA,
    ],
    **_SC_MESH,
)
def _gat_agg_kernel(xw2_hbm, ee_hbm, srcs_hbm, dsts_hbm, zeros_hbm,
                    num_hbm,
                    sidx_v, didx_v, rows0, rows1, ee0, ee1,
                    nacc_sh, gsem0, gsem1):
    cid, sid, wid = _wid()
    rows = pl.ds(sid * RPT, RPT)
    pltpu.sync_copy(zeros_hbm.at[rows], nacc_sh.at[rows])
    plsc.subcore_barrier()

    rowsb = (rows0, rows1)
    eeb = (ee0, ee1)
    gsem = (gsem0, gsem1)

    def cpseg(s):
        s = pl.multiple_of(s, SEGB)
        pltpu.sync_copy(srcs_hbm.at[wid, pl.ds(s, SEGB)], sidx_v)
        pltpu.sync_copy(dsts_hbm.at[wid, pl.ds(s, SEGB)], didx_v)

    def issue(jl, j, b):
        pltpu.async_copy(xw2_hbm.at[sidx_v.at[jl]], rowsb[b], gsem[b])
        pltpu.async_copy(ee_hbm.at[wid, j], eeb[b], gsem[b])

    def drain(b):
        pltpu.make_async_copy(xw2_hbm.at[sidx_v.at[0]], rowsb[b],
                              gsem[b]).wait()
        pltpu.make_async_copy(ee_hbm.at[0, 0], eeb[b], gsem[b]).wait()

    def process(jl, b):
        rb = rowsb[b]

        def scale(i4, carry):
            for u in range(4):
                i = i4 * 4 + u
                eer = eeb[b][i, pl.ds(0, 16)]
                for h in range(H):
                    s = eer[h]
                    for q in range(2):
                        sl = pl.ds(h * 32 + q * 16, 16)
                        rb[i, sl] = rb[i, sl] * s
            return carry

        lax.fori_loop(0, KB // 4, scale, 0)
        pltpu.sync_copy(rb, nacc_sh.at[didx_v.at[jl]], add=True)

    cpseg(0)
    issue(0, 0, 0)
    issue(1, 1, 1)

    def body(t, carry):
        j = 2 * t
        jl = lax.rem(j, SEGB)
        drain(0)
        process(jl, 0)
        drain(1)
        process(jl + 1, 1)

        @pl.when(j + 2 < NITB)
        def _():
            @pl.when(lax.rem(j + 2, SEGB) == 0)
            def _():
                cpseg(j + 2)
            issue(lax.rem(j + 2, SEGB), j + 2, 0)

        @pl.when(j + 3 < NITB)
        def _():
            issue(lax.rem(j + 3, SEGB), j + 3, 1)
        return carry

    lax.fori_loop(0, NITB // 2, body, 0)
    plsc.subcore_barrier()
    pltpu.sync_copy(nacc_sh.at[rows], num_hbm.at[cid, rows])


_B = 512
_GRID = NP // _B


def _full_spec(shape):
    return pl.BlockSpec(shape, lambda i: (0,) * len(shape))


def _row_spec(cols):
    return pl.BlockSpec((_B, cols), lambda i: (i, 0))


def _pair_spec(cols):
    return pl.BlockSpec((NC, _B, cols), lambda i: (0, i, 0))


def _tc_call(body, in_specs, out_specs, out_shapes):
    return pl.pallas_call(
        body,
        grid=(_GRID,),
        in_specs=in_specs,
        out_specs=out_specs,
        out_shape=out_shapes,
    )


def _k2_body(x_ref, w_ref, dg_ref, xw_ref, dv_ref):
    dv8 = lax.rsqrt(1.0 + dg_ref[0] + dg_ref[1])
    xw = jnp.dot(x_ref[...], w_ref[...], preferred_element_type=_f32)
    xw_ref[...] = xw * dv8[:, 0:1]
    dv_ref[...] = dv8


def _k4_body(a_ref, xwp_ref, dv_ref, b1_ref, wg_ref, asd_ref,
             xw2_ref, alsd_ref):
    dv = dv_ref[...][:, 0:1]
    h1 = jnp.maximum(
        dv * (a_ref[0] + a_ref[1] + xwp_ref[...]) + b1_ref[...], 0.0)
    xw2 = jnp.dot(h1, wg_ref[...], preferred_element_type=_f32)
    xw2_ref[...] = xw2
    alsd_ref[...] = jnp.dot(xw2, asd_ref[...], preferred_element_type=_f32)


def _k6_body(n_ref, q_ref, alsd_ref, xw2_ref,
             bg_ref, w3_ref, dv_ref, out_ref):
    al = alsd_ref[...]
    a = al[:, 0:4] + al[:, 4:8]
    ees = jnp.exp(jnp.where(a >= 0.0, a, 0.2 * a))
    den4 = q_ref[0][:, 0:4] + q_ref[1][:, 0:4] + ees + 1e-16
    hh = lax.broadcasted_iota(_i32, (4, D), 0)
    cc = lax.broadcasted_iota(_i32, (4, D), 1)
    sel = (cc // C == hh).astype(_f32)
    eef = jnp.dot(ees, sel, preferred_element_type=_f32)
    denf = jnp.dot(den4, sel, preferred_element_type=_f32)
    num = n_ref[0] + n_ref[1] + eef * xw2_ref[...]
    h2 = jnp.maximum(num / denf + bg_ref[...], 0.0)
    out_ref[...] = (jnp.dot(h2, w3_ref[...], preferred_element_type=_f32)
                    * dv_ref[...][:, 0:1])


_RS = float((1.0 + 1e-5) ** -0.5)


def _k8_body(r_ref, xwp_ref, dv_ref, b3_ref, wm1_ref, bm1_ref,
             gam_ref, bet_ref, wm2_ref, bm2_ref, wu_ref, bu_ref, tv_ref,
             p_ref, tu_ref):
    dv = dv_ref[...][:, 0:1]
    h3 = jnp.maximum(
        dv * (r_ref[0] + r_ref[1] + xwp_ref[...]) + b3_ref[...], 0.0)

    def mlp(h):
        z = jnp.dot(h, wm1_ref[...], preferred_element_type=_f32) + bm1_ref[...]
        z = z * _RS * gam_ref[...] + bet_ref[...]
        z = jnp.maximum(z, 0.0)
        return jnp.dot(z, wm2_ref[...], preferred_element_type=_f32) + bm2_ref[...]

    def softmax(logits):
        l = logits / TEMP
        m = jnp.max(l, axis=1, keepdims=True)
        ex = jnp.exp(l - m)
        return ex / jnp.sum(ex, axis=1, keepdims=True)

    tv = tv_ref[...]
    p1 = softmax(mlp(h3))
    us = jnp.dot(p1 * tv, wu_ref[...], preferred_element_type=_f32) + bu_ref[...]
    p2 = softmax(mlp(h3 + us))
    p_ref[...] = p2[:, 0:T]
    tu_ref[...] = jnp.sum(p2 * tv, axis=1, keepdims=True)


def kernel(x, edge_index, W1, b1, Wg, att_src, att_dst, bg, W3, b3,
           Wm1, bm1, gamma, beta, Wm2, bm2, Wu, bu, type_unc):
    pad = jnp.full((NW, EPWP - EPW), N, _i32)
    srcp = jnp.concatenate([edge_index[0].reshape(NW, EPW), pad], axis=1)
    dstp = jnp.concatenate([edge_index[1].reshape(NW, EPW), pad], axis=1)
    srcsE = srcp.reshape(NW, NITE, KE)
    dstsE = dstp.reshape(NW, NITE, KE)
    srcsB = srcp.reshape(NW, NITB, KB)
    dstsB = dstp.reshape(NW, NITB, KB)
    dsts80 = edge_index[1].reshape(NW, NITD, KD)
    xp = jnp.zeros((NP, D), _f32).at[:N].set(x)

    zerosND = jnp.zeros((NP, D), _f32)
    zerosN8 = jnp.zeros((NP, 8), _f32)
    zerosN16 = jnp.zeros((NP, 16), _f32)
    zerosKB16 = jnp.zeros((KB, 16), _f32)
    onesK8 = jnp.ones((KD, 8), _f32)

    eye = jnp.eye(H, dtype=_f32)
    Asd = jnp.concatenate(
        [(att_src[:, :, None] * eye[:, None, :]).reshape(D, H),
         (att_dst[:, :, None] * eye[:, None, :]).reshape(D, H)], axis=1)

    Wm2p = jnp.zeros((D, D), _f32).at[:, :T].set(Wm2)
    bm2p = jnp.full((1, D), -1e30, _f32).at[0, :T].set(bm2)
    Wup = jnp.zeros((D, D), _f32).at[:T, :].set(Wu)
    tvp = jnp.zeros((1, D), _f32).at[0, :T].set(type_unc)

    b1r = b1.reshape(1, D)
    bgr = bg.reshape(1, D)
    b3r = b3.reshape(1, D)
    bm1r = bm1.reshape(1, D)
    gammar = gamma.reshape(1, D)
    betar = beta.reshape(1, D)
    bur = bu.reshape(1, D)

    deg = _deg_kernel(dsts80, onesK8, zerosN8)

    xw1p, dinv8 = _tc_call(
        _k2_body,
        [_row_spec(D), _full_spec((D, D)), _pair_spec(8)],
        [_row_spec(D), _row_spec(8)],
        [jax.ShapeDtypeStruct((NP, D), _f32), jax.ShapeDtypeStruct((NP, 8), _f32)],
    )(xp, W1, deg)

    agg1 = _gcn_kernel(xw1p, srcsE, dstsE, zerosND)

    xw2, alsd = _tc_call(
        _k4_body,
        [_pair_spec(D), _row_spec(D), _row_spec(8),
         _full_spec((1, D)), _full_spec((D, D)), _full_spec((D, 2 * H))],
        [_row_spec(D), _row_spec(2 * H)],
        [jax.ShapeDtypeStruct((NP, D), _f32),
         jax.ShapeDtypeStruct((NP, 2 * H), _f32)],
    )(agg1, xw1p, dinv8, b1r, Wg, Asd)

    ee = _gat_att_kernel(alsd.reshape(NP * 8), srcsB, dstsB, zerosKB16)
    q = _den_kernel(ee.reshape(NW, NITE, KE, 16), dstsE, zerosN16)
    num = _gat_agg_kernel(xw2, ee, srcsB, dstsB, zerosND)

    xw3p = _tc_call(
        _k6_body,
        [_pair_spec(D), _pair_spec(16),
         _row_spec(2 * H), _row_spec(D),
         _full_spec((1, D)), _full_spec((D, D)), _row_spec(8)],
        _row_spec(D),
        jax.ShapeDtypeStruct((NP, D), _f32),
    )(num, q, alsd, xw2, bgr, W3, dinv8)

    agg3 = _gcn_kernel(xw3p, srcsE, dstsE, zerosND)

    p, tu = _tc_call(
        _k8_body,
        [_pair_spec(D), _row_spec(D), _row_spec(8),
         _full_spec((1, D)), _full_spec((D, D)), _full_spec((1, D)),
         _full_spec((1, D)), _full_spec((1, D)), _full_spec((D, D)),
         _full_spec((1, D)), _full_spec((D, D)), _full_spec((1, D)),
         _full_spec((1, D))],
        [_row_spec(T), _row_spec(1)],
        [jax.ShapeDtypeStruct((NP, T), _f32), jax.ShapeDtypeStruct((NP, 1), _f32)],
    )(agg3, xw3p, dinv8, b3r, Wm1, bm1r, gammar, betar, Wm2p, bm2p,
      Wup, bur, tvp)

    return (p[:N], tu[:N])
